# Initial kernel scaffold; baseline (speedup 1.0000x reference)
import jax
import jax.numpy as jnp
from jax.experimental import pallas as pl


def kernel(x, pos, g0_x, g0_edge_index, g0_edge_embed, g0_pos, g1_x, g1_edge_index, g1_edge_embed, g1_pos, Wl0, bl0, gl0, betal0, Wg0, bg0, gg0, betag0, Wl1, bl1, gl1, betal1, Wg1, bg1, gg1, betag1):
    raise NotImplementedError("write your pallas kernel here")


# speedup vs baseline: 1.3130x; 1.3130x over previous
import functools
import jax
import jax.numpy as jnp
from jax.experimental import pallas as pl
from jax.experimental.pallas import tpu as pltpu

NN = 10000
NE = 320000
DF = 128
DEDGE = 16
DL = 128
DG = 128
EPS = 1e-5


def _k_ab(gx_ref, gpos_ref, pos_ref, wx_ref, wr_ref, a_ref, b_ref):
    wx = wx_ref[...]
    wr = wr_ref[...]
    a = jnp.dot(gx_ref[...], wx, preferred_element_type=jnp.float32)
    gp = gpos_ref[...]
    p = pos_ref[...]
    for k in range(3):
        a = a + gp[:, k:k + 1] * wr[k:k + 1, :]
    b = p[:, 0:1] * wr[0:1, :]
    for k in range(1, 3):
        b = b + p[:, k:k + 1] * wr[k:k + 1, :]
    a_ref[...] = a
    b_ref[...] = b


@jax.jit
def _make_ab(gx, gpos, pos, wx, wr):
    return pl.pallas_call(
        _k_ab,
        out_shape=(
            jax.ShapeDtypeStruct((NN, DL), jnp.float32),
            jax.ShapeDtypeStruct((NN, DL), jnp.float32),
        ),
    )(gx, gpos, pos, wx, wr)


_BE = 4000


def _k_e(ee_ref, we_ref, e_ref):
    e_ref[...] = jnp.dot(ee_ref[...], we_ref[...],
                         preferred_element_type=jnp.float32)


@jax.jit
def _make_e(ee, we):
    return pl.pallas_call(
        _k_e,
        grid=(NE // _BE,),
        in_specs=[
            pl.BlockSpec((_BE, DEDGE), lambda i: (i, 0)),
            pl.BlockSpec((DEDGE, DL), lambda i: (0, 0)),
        ],
        out_specs=pl.BlockSpec((_BE, DL), lambda i: (i, 0)),
        out_shape=jax.ShapeDtypeStruct((NE, DL), jnp.float32),
    )(ee, we)


def _k_comb(m0_ref, m1_ref, s0_ref, s1_ref, cnt_ref, ssq_ref, b_ref,
            gl_ref, betal_ref, wg_ref, bg_ref, gg_ref, betag_ref, out_ref):
    b = b_ref[...]
    s = s0_ref[...] + s1_ref[...]
    cnt = cnt_ref[...]
    sum_zp = jnp.sum(s, axis=0, keepdims=True)
    q1 = jnp.sum(cnt * b, axis=0, keepdims=True)
    q2 = jnp.sum(cnt * b * b, axis=0, keepdims=True)
    cross = jnp.sum(s * b, axis=0, keepdims=True)
    ssq = jnp.sum(ssq_ref[...], axis=0, keepdims=True)
    inv_ne = 1.0 / NE
    mean = (sum_zp - q1) * inv_ne
    ex2 = (ssq - 2.0 * cross + q2) * inv_ne
    var = ex2 - mean * mean
    mz = jnp.maximum(m0_ref[...], m1_ref[...]) - b
    normed = (mz - mean) * jax.lax.rsqrt(var + EPS) * gl_ref[...] \
        + betal_ref[...]
    agg = jnp.where(jnp.isfinite(mz), jnp.maximum(normed, 0.0), 0.0)
    y = jnp.dot(agg, wg_ref[...], preferred_element_type=jnp.float32) \
        + bg_ref[...]
    my = jnp.mean(y, axis=0, keepdims=True)
    vy = jnp.mean(y * y, axis=0, keepdims=True) - my * my
    yn = (y - my) * jax.lax.rsqrt(vy + EPS) * gg_ref[...] + betag_ref[...]
    out_ref[...] = jnp.maximum(yn, 0.0)


@jax.jit
def _combine(m0, m1, s0, s1, cnt, ssq, b, gl, betal, wg, bg, gg, betag):
    return pl.pallas_call(
        _k_comb,
        out_shape=jax.ShapeDtypeStruct((NN, DG), jnp.float32),
    )(m0, m1, s0, s1, cnt, ssq, b,
      gl.reshape(1, DL), betal.reshape(1, DL), wg, bg.reshape(1, DG),
      gg.reshape(1, DG), betag.reshape(1, DG))


def _middle(a, e, src, dst):
    zp = a[src] + e
    m = jax.ops.segment_max(zp, dst, num_segments=NN)
    s = jax.ops.segment_sum(zp, dst, num_segments=NN)
    cnt = jax.ops.segment_sum(jnp.ones((NE,), jnp.float32), dst,
                              num_segments=NN)
    ssq = jnp.sum(zp * zp, axis=0, keepdims=True)
    return m, s, cnt.reshape(NN, 1), ssq


def _conv(xq_pos, gx, gei, gee, gpos, Wl, gl, betal, Wg, bg, gg, betag):
    wx = Wl[0:DF, :]
    we = Wl[DF:DF + DEDGE, :]
    wr = Wl[DF + DEDGE:, :]
    a, b = _make_ab(gx, gpos, xq_pos, wx, wr)
    e = _make_e(gee, we)
    m, s, cnt, ssq = _middle(a, e, gei[0], gei[1])
    neg = jnp.full((NN, DL), -jnp.inf, jnp.float32)
    zero = jnp.zeros((NN, DL), jnp.float32)
    return _combine(m, neg, s, zero, cnt, ssq, b, gl, betal, Wg, bg, gg,
                    betag)


def kernel(x, pos, g0_x, g0_edge_index, g0_edge_embed, g0_pos, g1_x,
           g1_edge_index, g1_edge_embed, g1_pos, Wl0, bl0, gl0, betal0,
           Wg0, bg0, gg0, betag0, Wl1, bl1, gl1, betal1, Wg1, bg1, gg1,
           betag1):
    o0 = _conv(pos, g0_x, g0_edge_index, g0_edge_embed, g0_pos, Wl0, gl0,
               betal0, Wg0, bg0, gg0, betag0)
    o1 = _conv(pos, g1_x, g1_edge_index, g1_edge_embed, g1_pos, Wl1, gl1,
               betal1, Wg1, bg1, gg1, betag1)
    return jnp.concatenate([o0, o1], axis=-1)


# speedup vs baseline: 1.6349x; 1.2452x over previous
import functools
import jax
import jax.numpy as jnp
from jax import lax
from jax.experimental import pallas as pl
from jax.experimental.pallas import tpu as pltpu
from jax.experimental.pallas import tpu_sc as plsc

NN = 10000
NE = 320000
DF = 128
DEDGE = 16
DL = 128
DG = 128
EPS = 1e-5


def _k_ab(gx_ref, gpos_ref, pos_ref, wx_ref, wr_ref, a_ref, b_ref):
    wx = wx_ref[...]
    wr = wr_ref[...]
    a = jnp.dot(gx_ref[...], wx, preferred_element_type=jnp.float32)
    gp = gpos_ref[...]
    p = pos_ref[...]
    for k in range(3):
        a = a + gp[:, k:k + 1] * wr[k:k + 1, :]
    b = p[:, 0:1] * wr[0:1, :]
    for k in range(1, 3):
        b = b + p[:, k:k + 1] * wr[k:k + 1, :]
    a_ref[...] = a
    b_ref[...] = b


@jax.jit
def _make_ab(gx, gpos, pos, wx, wr):
    return pl.pallas_call(
        _k_ab,
        out_shape=(
            jax.ShapeDtypeStruct((NN, DL), jnp.float32),
            jax.ShapeDtypeStruct((NN, DL), jnp.float32),
        ),
    )(gx, gpos, pos, wx, wr)


_BE = 4000


def _k_e(ee_ref, we_ref, e_ref):
    e_ref[...] = jnp.dot(ee_ref[...], we_ref[...],
                         preferred_element_type=jnp.float32)


@jax.jit
def _make_e(ee, we):
    return pl.pallas_call(
        _k_e,
        grid=(NE // _BE,),
        in_specs=[
            pl.BlockSpec((_BE, DEDGE), lambda i: (i, 0)),
            pl.BlockSpec((DEDGE, DL), lambda i: (0, 0)),
        ],
        out_specs=pl.BlockSpec((_BE, DL), lambda i: (i, 0)),
        out_shape=jax.ShapeDtypeStruct((NE, DL), jnp.float32),
    )(ee, we)


def _k_comb(m_ref, s_ref, cnt_ref, ssq_ref, b_ref,
            gl_ref, betal_ref, wg_ref, bg_ref, gg_ref, betag_ref, out_ref):
    b = b_ref[...]
    s = s_ref[...]
    cnt = cnt_ref[...]
    sum_zp = jnp.sum(s, axis=0, keepdims=True)
    q1 = jnp.sum(cnt * b, axis=0, keepdims=True)
    q2 = jnp.sum(cnt * b * b, axis=0, keepdims=True)
    cross = jnp.sum(s * b, axis=0, keepdims=True)
    ssq = jnp.sum(ssq_ref[...], axis=0, keepdims=True)
    inv_ne = 1.0 / NE
    mean = (sum_zp - q1) * inv_ne
    ex2 = (ssq - 2.0 * cross + q2) * inv_ne
    var = ex2 - mean * mean
    mz = m_ref[...] - b
    normed = (mz - mean) * jax.lax.rsqrt(var + EPS) * gl_ref[...] \
        + betal_ref[...]
    agg = jnp.where(jnp.isfinite(mz), jnp.maximum(normed, 0.0), 0.0)
    y = jnp.dot(agg, wg_ref[...], preferred_element_type=jnp.float32) \
        + bg_ref[...]
    my = jnp.mean(y, axis=0, keepdims=True)
    vy = jnp.mean(y * y, axis=0, keepdims=True) - my * my
    yn = (y - my) * jax.lax.rsqrt(vy + EPS) * gg_ref[...] + betag_ref[...]
    out_ref[...] = jnp.maximum(yn, 0.0)


@jax.jit
def _combine(m, s, cnt, ssq, b, gl, betal, wg, bg, gg, betag):
    return pl.pallas_call(
        _k_comb,
        out_shape=jax.ShapeDtypeStruct((NN, DG), jnp.float32),
    )(m, s, cnt, ssq, b,
      gl.reshape(1, DL), betal.reshape(1, DL), wg, bg.reshape(1, DG),
      gg.reshape(1, DG), betag.reshape(1, DG))


ECH = NE // 32
NB = 32
BSEG = NN // NB + 1
SEGP = 320
CAP = 448
CH = 32

_sc_mesh = plsc.VectorSubcoreMesh(core_axis_name="c", subcore_axis_name="s",
                                  num_cores=2, num_subcores=16)


def _scan_body(src_hbm, dst_hbm, xall, xc, src_loc, dst_loc, posb, packes,
               cbuf, ptrs):
    c = lax.axis_index("c")
    s = lax.axis_index("s")
    w = c * 16 + s
    ebase = w * ECH
    i16 = lax.iota(jnp.int32, 16)
    z16i = jnp.zeros((16,), jnp.int32)

    pltpu.sync_copy(src_hbm.at[pl.ds(ebase, ECH)], src_loc)
    pltpu.sync_copy(dst_hbm.at[pl.ds(ebase, ECH)], dst_loc)

    def _zp(i, _):
        posb[pl.ds(i * 16, 16)] = z16i
        return 0
    lax.fori_loop(0, NB * CAP // 16, _zp, 0)
    for b in range(NB):
        ptrs[b] = 0

    def _scan(i, _):
        dvec = dst_loc[pl.ds(i * 16, 16)]
        bvec = ((dvec.astype(jnp.float32) + 0.5)
                * (1.0 / BSEG)).astype(jnp.int32)
        pvec = i16 + i * 16
        for b in range(NB):
            mask = bvec == b
            p = ptrs[b]
            cs = plsc.cumsum(mask.astype(jnp.int32))
            idx = jnp.maximum(b * CAP + p + cs - 1, 0)
            plsc.store_scatter(posb, [idx], pvec, mask=mask)
            ptrs[b] = jnp.minimum(p + cs[15], CAP - 16)
        return 0
    lax.fori_loop(0, ECH // 16, _scan, 0)

    for b in range(NB):
        def _pk(g, _):
            pv = posb[pl.ds(b * CAP + g * 16, 16)]
            sv = plsc.load_gather(src_loc, [pv])
            dv = plsc.load_gather(dst_loc, [pv])
            packes[pl.ds(g * 16, 16)] = pv + ebase
            packes[pl.ds(CAP + g * 16, 16)] = sv
            packes[pl.ds(2 * CAP + g * 16, 16)] = dv
            return 0
        lax.fori_loop(0, CAP // 16, _pk, 0)
        pltpu.sync_copy(packes, xall.at[w, b])
    for half in range(2):
        cvec = z16i
        for b in range(16):
            cvec = jnp.where(i16 == b, ptrs[16 * half + b], cvec)
        cbuf[pl.ds(16 * half, 16)] = cvec
    pltpu.sync_copy(cbuf, xc.at[w])


_sc_scan = functools.partial(
    pl.kernel,
    mesh=_sc_mesh,
    compiler_params=pltpu.CompilerParams(needs_layout_passes=False),
    out_type=[
        jax.ShapeDtypeStruct((NB, NB, 3 * CAP), jnp.int32),
        jax.ShapeDtypeStruct((NB, NB), jnp.int32),
    ],
    scratch_types=[
        pltpu.VMEM((ECH,), jnp.int32),
        pltpu.VMEM((ECH,), jnp.int32),
        pltpu.VMEM((NB * CAP,), jnp.int32),
        pltpu.VMEM((3 * CAP,), jnp.int32),
        pltpu.VMEM((NB,), jnp.int32),
        pltpu.SMEM((NB,), jnp.int32),
    ],
)(_scan_body)


def _consume_body(a_hbm, e_hbm, xall, xc, m_out, s_out, cnt_out, ssq_out,
                  proces, cntx, abuf, ebuf, mloc, sloc, cntl, ssqa,
                  semA, semB):
    c = lax.axis_index("c")
    s = lax.axis_index("s")
    w = c * 16 + s
    nbase = w * BSEG
    i16 = lax.iota(jnp.int32, 16)
    z16f = jnp.zeros((16,), jnp.float32)
    ninf16 = jnp.full((16,), -jnp.inf, jnp.float32)
    wlane = jnp.bitwise_and(w, 15)
    whalf = lax.shift_right_logical(w, 4)

    def _mz(i, _):
        mloc[i >> 3, pl.ds((i & 7) * 16, 16)] = ninf16
        sloc[i >> 3, pl.ds((i & 7) * 16, 16)] = z16f
        return 0
    lax.fori_loop(0, SEGP * 8, _mz, 0)

    def _cz(i, _):
        cntl[pl.ds(i * 16, 16)] = z16f
        return 0
    lax.fori_loop(0, SEGP // 16, _cz, 0)
    for k in range(8):
        ssqa[pl.ds(k * 16, 16)] = z16f
    pltpu.sync_copy(xc, cntx)

    def _per_q(q, _):
        pltpu.sync_copy(xall.at[q, w], proces)
        crow = cntx[q, pl.ds(whalf * 16, 16)]
        cq = jnp.sum(jnp.where(i16 == wlane, crow, 0))
        nfull = lax.shift_right_logical(cq, 5)
        tail = jnp.bitwise_and(cq, 31)

        def _edge(j2, dloc, accs):
            for k in range(8):
                av = abuf[j2, pl.ds(k * 16, 16)]
                ev = ebuf[j2, pl.ds(k * 16, 16)]
                z = av + ev
                if accs is None:
                    sv = ssqa[pl.ds(k * 16, 16)]
                    ssqa[pl.ds(k * 16, 16)] = sv + z * z
                else:
                    accs[k] = accs[k] + z * z
                mv = mloc[dloc, pl.ds(k * 16, 16)]
                mloc[dloc, pl.ds(k * 16, 16)] = jnp.maximum(mv, z)
                sv2 = sloc[dloc, pl.ds(k * 16, 16)]
                sloc[dloc, pl.ds(k * 16, 16)] = sv2 + z
            grp = lax.shift_right_logical(dloc, 4)
            lane = jnp.bitwise_and(dloc, 15)
            cv = cntl[pl.ds(grp * 16, 16)]
            cntl[pl.ds(grp * 16, 16)] = cv + jnp.where(i16 == lane, 1.0,
                                                       0.0)

        def _chunk(j, _):
            ca = pltpu.async_copy(
                a_hbm.at[proces.at[pl.ds(CAP + j * CH, CH)]], abuf, semA)
            ce = pltpu.async_copy(
                e_hbm.at[proces.at[pl.ds(j * CH, CH)]], ebuf, semB)
            ca.wait()
            ce.wait()
            acc = [z16f] * 8
            dvrows = [proces[pl.ds(2 * CAP + j * CH, 16)],
                      proces[pl.ds(2 * CAP + j * CH + 16, 16)]]
            for j2 in range(CH):
                dloc = dvrows[j2 // 16][j2 % 16] - nbase
                _edge(j2, dloc, acc)
            for k in range(8):
                sv = ssqa[pl.ds(k * 16, 16)]
                ssqa[pl.ds(k * 16, 16)] = sv + acc[k]
            return 0
        lax.fori_loop(0, nfull, _chunk, 0)

        @pl.when(tail > 0)
        def _():
            ca = pltpu.async_copy(
                a_hbm.at[proces.at[pl.ds(CAP + nfull * CH, CH)]], abuf,
                semA)
            ce = pltpu.async_copy(
                e_hbm.at[proces.at[pl.ds(nfull * CH, CH)]], ebuf, semB)
            ca.wait()
            ce.wait()

            def _tedge(j2, _):
                hv = proces[pl.ds(
                    2 * CAP + nfull * CH
                    + lax.shift_right_logical(j2, 4) * 16, 16)]
                dloc = jnp.sum(
                    jnp.where(i16 == jnp.bitwise_and(j2, 15), hv,
                              0)) - nbase
                _edge(j2, dloc, None)
                return 0
            lax.fori_loop(0, tail, _tedge, 0)
        return 0
    lax.fori_loop(0, NB, _per_q, 0)

    pltpu.sync_copy(mloc, m_out.at[w])
    pltpu.sync_copy(sloc, s_out.at[w])
    pltpu.sync_copy(cntl, cnt_out.at[w])
    pltpu.sync_copy(ssqa, ssq_out.at[w])


_sc_consume = functools.partial(
    pl.kernel,
    mesh=_sc_mesh,
    compiler_params=pltpu.CompilerParams(needs_layout_passes=False),
    out_type=[
        jax.ShapeDtypeStruct((NB, SEGP, DL), jnp.float32),
        jax.ShapeDtypeStruct((NB, SEGP, DL), jnp.float32),
        jax.ShapeDtypeStruct((NB, SEGP), jnp.float32),
        jax.ShapeDtypeStruct((NB, DL), jnp.float32),
    ],
    scratch_types=[
        pltpu.VMEM((3 * CAP,), jnp.int32),
        pltpu.VMEM((NB, NB), jnp.int32),
        pltpu.VMEM((CH, DL), jnp.float32),
        pltpu.VMEM((CH, DL), jnp.float32),
        pltpu.VMEM((SEGP, DL), jnp.float32),
        pltpu.VMEM((SEGP, DL), jnp.float32),
        pltpu.VMEM((SEGP,), jnp.float32),
        pltpu.VMEM((DL,), jnp.float32),
        pltpu.SemaphoreType.DMA,
        pltpu.SemaphoreType.DMA,
    ],
)(_consume_body)


def _conv(xq_pos, gx, gei, gee, gpos, Wl, gl, betal, Wg, bg, gg, betag,
          xall, xc):
    wx = Wl[0:DF, :]
    we = Wl[DF:DF + DEDGE, :]
    wr = Wl[DF + DEDGE:, :]
    a, b = _make_ab(gx, gpos, xq_pos, wx, wr)
    e = _make_e(gee, we)
    m_p, s_p, cnt_p, ssq_p = _sc_consume(a, e, xall, xc)
    m = m_p[:, :BSEG].reshape(NB * BSEG, DL)[:NN]
    s = s_p[:, :BSEG].reshape(NB * BSEG, DL)[:NN]
    cnt = cnt_p[:, :BSEG].reshape(NB * BSEG, 1)[:NN]
    return _combine(m, s, cnt, ssq_p, b, gl, betal, Wg, bg, gg, betag)


def kernel(x, pos, g0_x, g0_edge_index, g0_edge_embed, g0_pos, g1_x,
           g1_edge_index, g1_edge_embed, g1_pos, Wl0, bl0, gl0, betal0,
           Wg0, bg0, gg0, betag0, Wl1, bl1, gl1, betal1, Wg1, bg1, gg1,
           betag1):
    x0, c0 = _sc_scan(g0_edge_index[0], g0_edge_index[1])
    x1, c1 = _sc_scan(g1_edge_index[0], g1_edge_index[1])
    o0 = _conv(pos, g0_x, g0_edge_index, g0_edge_embed, g0_pos, Wl0, gl0,
               betal0, Wg0, bg0, gg0, betag0, x0, c0)
    o1 = _conv(pos, g1_x, g1_edge_index, g1_edge_embed, g1_pos, Wl1, gl1,
               betal1, Wg1, bg1, gg1, betag1, x1, c1)
    return jnp.concatenate([o0, o1], axis=-1)
